# baseline (speedup 1.0000x reference)
import jax
import jax.numpy as jnp
from jax import lax
from jax.experimental import pallas as pl
from jax.experimental.pallas import tpu as pltpu
from jax.experimental.pallas import tpu_sc as plsc

N = 100000
NW = 32
CH = 3136
STEPS = CH // 16


def _sc_body(coords_hbm, table_hbm, bid_hbm, out_hbm,
             coords_v, table_v, bid_v, out_v):
    c = lax.axis_index("c")
    s = lax.axis_index("s")
    wid = s * 2 + c
    base = jnp.minimum(wid * CH, N - CH)

    pltpu.sync_copy(table_hbm, table_v)
    pltpu.sync_copy(bid_hbm.at[pl.ds(base, CH)], bid_v)
    pltpu.sync_copy(coords_hbm.at[pl.ds(base * 3, CH * 3)], coords_v)

    iota3 = lax.iota(jnp.int32, 16) * 3

    @plsc.parallel_loop(0, STEPS, unroll=4)
    def step(sidx):
        o = sidx * 16
        b16 = bid_v[pl.ds(o, 16)]
        t = b16 * 9
        ci = iota3 + o * 3
        rx = plsc.load_gather(coords_v, [ci])
        ry = plsc.load_gather(coords_v, [ci + 1])
        rz = plsc.load_gather(coords_v, [ci + 2])
        m = [plsc.load_gather(table_v, [t + k]) for k in range(9)]
        ox = rx * m[0] + ry * m[3] + rz * m[6]
        oy = rx * m[1] + ry * m[4] + rz * m[7]
        oz = rx * m[2] + ry * m[5] + rz * m[8]
        plsc.store_scatter(out_v, [ci], ox)
        plsc.store_scatter(out_v, [ci + 1], oy)
        plsc.store_scatter(out_v, [ci + 2], oz)
    pltpu.sync_copy(out_v, out_hbm.at[pl.ds(base * 3, CH * 3)])


def kernel(real_coordinates, inv_lattice_matrices, batch_id):
    coords_flat = real_coordinates.reshape(-1)
    table = inv_lattice_matrices.reshape(-1)
    bid = batch_id.astype(jnp.int32)
    mesh = plsc.VectorSubcoreMesh(core_axis_name="c", subcore_axis_name="s")
    out_flat = pl.kernel(
        _sc_body,
        out_type=jax.ShapeDtypeStruct((N * 3,), jnp.float32),
        mesh=mesh,
        scratch_types=[
            pltpu.VMEM((CH * 3,), jnp.float32),
            pltpu.VMEM((table.shape[0],), jnp.float32),
            pltpu.VMEM((CH,), jnp.int32),
            pltpu.VMEM((CH * 3,), jnp.float32),
        ],
        compiler_params=pltpu.CompilerParams(
            needs_layout_passes=False,
            disable_bounds_checks=True,
            disable_semaphore_checks=True,
            skip_device_barrier=True,
        ),
    )(coords_flat, table, bid)
    return out_flat.reshape(N, 3)


# speedup vs baseline: 5.2801x; 5.2801x over previous
import jax
import jax.numpy as jnp
from jax import lax
from jax.experimental import pallas as pl
from jax.experimental.pallas import tpu as pltpu
from jax.experimental.pallas import tpu_sc as plsc

N = 100000
NW = 32
CH = 3136
STEPS = CH // 16


def _sc_body(coords_hbm, table_hbm, bid_hbm, out_hbm,
             bid_v, table_v, cx, cy, cz, ox_v, oy_v, oz_v):
    c = lax.axis_index("c")
    s = lax.axis_index("s")
    wid = s * 2 + c
    base = jnp.minimum(wid * CH, N - CH)

    pltpu.sync_copy(table_hbm, table_v)
    pltpu.sync_copy(bid_hbm.at[pl.ds(base, CH)], bid_v)
    pltpu.sync_copy(coords_hbm.at[pl.ds(base, CH)], cx)
    pltpu.sync_copy(coords_hbm.at[pl.ds(N + base, CH)], cy)
    pltpu.sync_copy(coords_hbm.at[pl.ds(2 * N + base, CH)], cz)

    @plsc.parallel_loop(0, STEPS, unroll=4)
    def step(sidx):
        o = sidx * 16
        b16 = bid_v[pl.ds(o, 16)]
        rx = cx[pl.ds(o, 16)]
        ry = cy[pl.ds(o, 16)]
        rz = cz[pl.ds(o, 16)]
        m = [plsc.load_gather(table_v, [b16 + k * 256]) for k in range(9)]
        ox_v[pl.ds(o, 16)] = rx * m[0] + ry * m[3] + rz * m[6]
        oy_v[pl.ds(o, 16)] = rx * m[1] + ry * m[4] + rz * m[7]
        oz_v[pl.ds(o, 16)] = rx * m[2] + ry * m[5] + rz * m[8]

    pltpu.sync_copy(ox_v, out_hbm.at[pl.ds(base, CH)])
    pltpu.sync_copy(oy_v, out_hbm.at[pl.ds(N + base, CH)])
    pltpu.sync_copy(oz_v, out_hbm.at[pl.ds(2 * N + base, CH)])


def kernel(real_coordinates, inv_lattice_matrices, batch_id):
    coords_t = real_coordinates.T.reshape(-1)
    table_planes = jnp.transpose(inv_lattice_matrices, (1, 2, 0)).reshape(-1)
    bid = batch_id.astype(jnp.int32)
    mesh = plsc.VectorSubcoreMesh(core_axis_name="c", subcore_axis_name="s")
    out_t = pl.kernel(
        _sc_body,
        out_type=jax.ShapeDtypeStruct((3 * N,), jnp.float32),
        mesh=mesh,
        scratch_types=[
            pltpu.VMEM((CH,), jnp.int32),
            pltpu.VMEM((9 * 256,), jnp.float32),
            pltpu.VMEM((CH,), jnp.float32),
            pltpu.VMEM((CH,), jnp.float32),
            pltpu.VMEM((CH,), jnp.float32),
            pltpu.VMEM((CH,), jnp.float32),
            pltpu.VMEM((CH,), jnp.float32),
            pltpu.VMEM((CH,), jnp.float32),
        ],
        compiler_params=pltpu.CompilerParams(
            needs_layout_passes=False,
            disable_bounds_checks=True,
            disable_semaphore_checks=True,
            skip_device_barrier=True,
        ),
    )(coords_t, table_planes, bid)
    return out_t.reshape(3, N).T
